# baseline (speedup 1.0000x reference)
import functools

import jax
import jax.numpy as jnp
from jax.experimental import pallas as pl
from jax.experimental.pallas import tpu as pltpu

_SEQ_PAD = 256


def _body(user_ref, items_ref, q_ref, emb_hbm, out_ref, emb_vmem, sem):
    pltpu.make_async_copy(
        emb_hbm.at[pl.ds(user_ref[0], 1)], emb_vmem, sem).start()
    items_v = items_ref[0, :]
    n_items = q_ref.shape[0]
    col = jax.lax.broadcasted_iota(jnp.int32, (_SEQ_PAD, n_items), 1)
    onehot = (col == items_v[:, None]).astype(jnp.float32)
    qf = q_ref[...].astype(jnp.float32)
    q_rows = jnp.dot(onehot, qf, preferred_element_type=jnp.float32)
    pltpu.make_async_copy(
        emb_hbm.at[pl.ds(user_ref[0], 1)], emb_vmem, sem).wait()
    emb = emb_vmem[0]
    out_ref[...] = jnp.dot(q_rows, emb, preferred_element_type=jnp.float32)


def kernel(user, Q_matrix, items, skill_embedding):
    seq_len = items.shape[0]
    n_items, skill_num = Q_matrix.shape
    k_hidden = skill_embedding.shape[2]
    user_arr = jnp.asarray(user, jnp.int32).reshape(1)
    items_pad = jnp.zeros((1, _SEQ_PAD), jnp.int32).at[0, :seq_len].set(
        items.astype(jnp.int32))

    grid_spec = pltpu.PrefetchScalarGridSpec(
        num_scalar_prefetch=1,
        grid=(1,),
        in_specs=[
            pl.BlockSpec((1, _SEQ_PAD), lambda i, u: (0, 0)),
            pl.BlockSpec((n_items, skill_num), lambda i, u: (0, 0)),
            pl.BlockSpec(memory_space=pl.ANY),
        ],
        out_specs=pl.BlockSpec((_SEQ_PAD, k_hidden), lambda i, u: (0, 0)),
        scratch_shapes=[
            pltpu.VMEM((1, skill_num, k_hidden), jnp.float32),
            pltpu.SemaphoreType.DMA,
        ],
    )
    out = pl.pallas_call(
        _body,
        grid_spec=grid_spec,
        out_shape=jax.ShapeDtypeStruct((_SEQ_PAD, k_hidden), jnp.float32),
    )(user_arr, items_pad, Q_matrix, skill_embedding)
    return out[:seq_len]


# speedup vs baseline: 51.3202x; 51.3202x over previous
import jax
import jax.numpy as jnp
from jax.experimental import pallas as pl
from jax.experimental.pallas import tpu as pltpu

_SEQ_PAD = 256


def _body(items_ref, q_ref, emb_ref, out_ref):
    items_v = items_ref[0, :]
    n_items = q_ref.shape[0]
    col = jax.lax.broadcasted_iota(jnp.int32, (_SEQ_PAD, n_items), 1)
    onehot = (col == items_v[:, None]).astype(jnp.float32)
    qf = q_ref[...].astype(jnp.float32)
    q_rows = jnp.dot(onehot, qf, preferred_element_type=jnp.float32)
    out_ref[...] = jnp.dot(q_rows, emb_ref[...],
                           preferred_element_type=jnp.float32)


def kernel(user, Q_matrix, items, skill_embedding):
    seq_len = items.shape[0]
    n_items, skill_num = Q_matrix.shape
    k_hidden = skill_embedding.shape[2]
    emb = jax.lax.dynamic_slice_in_dim(
        skill_embedding, jnp.asarray(user, jnp.int32), 1, axis=0)[0]
    items_pad = jnp.zeros((1, _SEQ_PAD), jnp.int32).at[0, :seq_len].set(
        items.astype(jnp.int32))
    out = pl.pallas_call(
        _body,
        out_shape=jax.ShapeDtypeStruct((_SEQ_PAD, k_hidden), jnp.float32),
    )(items_pad, Q_matrix, emb)
    return out[:seq_len]


# speedup vs baseline: 133.3746x; 2.5989x over previous
import jax
import jax.numpy as jnp
from jax.experimental import pallas as pl
from jax.experimental.pallas import tpu as pltpu


def _body(user_ref, items_ref, q_ref, emb_hbm, out_ref, emb_vmem, sem):
    pltpu.make_async_copy(
        emb_hbm.at[pl.ds(user_ref[0], 1)], emb_vmem, sem).start()
    items_v = items_ref[...]
    seq_len = items_v.shape[0]
    n_items = q_ref.shape[0]
    col = jax.lax.broadcasted_iota(jnp.int32, (seq_len, n_items), 1)
    onehot = (col == items_v[:, None]).astype(jnp.float32)
    qf = q_ref[...].astype(jnp.float32)
    q_rows = jnp.dot(onehot, qf, preferred_element_type=jnp.float32)
    pltpu.make_async_copy(
        emb_hbm.at[pl.ds(user_ref[0], 1)], emb_vmem, sem).wait()
    emb_kt = emb_vmem[0]
    out_ref[...] = jax.lax.dot_general(
        emb_kt, q_rows, (((1,), (1,)), ((), ())),
        preferred_element_type=jnp.float32)


def kernel(user, Q_matrix, items, skill_embedding):
    seq_len = items.shape[0]
    n_items, skill_num = Q_matrix.shape
    k_hidden = skill_embedding.shape[2]
    user_arr = jnp.asarray(user, jnp.int32).reshape(1)
    emb_t = jnp.swapaxes(skill_embedding, 1, 2)

    grid_spec = pltpu.PrefetchScalarGridSpec(
        num_scalar_prefetch=1,
        grid=(1,),
        in_specs=[
            pl.BlockSpec((seq_len,), lambda i, u: (0,)),
            pl.BlockSpec((n_items, skill_num), lambda i, u: (0, 0)),
            pl.BlockSpec(memory_space=pl.ANY),
        ],
        out_specs=pl.BlockSpec((k_hidden, seq_len), lambda i, u: (0, 0)),
        scratch_shapes=[
            pltpu.VMEM((1, k_hidden, skill_num), jnp.float32),
            pltpu.SemaphoreType.DMA,
        ],
    )
    out_t = pl.pallas_call(
        _body,
        grid_spec=grid_spec,
        out_shape=jax.ShapeDtypeStruct((k_hidden, seq_len), jnp.float32),
    )(user_arr, items.astype(jnp.int32), Q_matrix, emb_t)
    return out_t.T
